# baseline (speedup 1.0000x reference)
import functools

import jax
import jax.numpy as jnp
from jax.experimental import pallas as pl
from jax.experimental.pallas import tpu as pltpu
from jax.experimental.pallas import tpu_sc as plsc

_N = 16384
_C = 1000
_BR = 1024
_NS = 2
_NB = _N // (_BR * _NS)
_K = _N - 15565

_I32_MIN = -2147483648
_I32_MAXP = 2147483647


def _sc_gather(flat, flat_idx):
    info = plsc.get_sparse_core_info()
    nc, ns = info.num_cores, info.num_subcores
    nw = nc * ns
    bpw = _N // nw

    mesh = plsc.VectorSubcoreMesh(core_axis_name="c", subcore_axis_name="s")

    @functools.partial(
        pl.kernel, mesh=mesh,
        out_type=jax.ShapeDtypeStruct((_N,), jnp.float32),
        scratch_types=[
            pltpu.VMEM((bpw,), jnp.int32),
            pltpu.VMEM((bpw,), jnp.float32),
            pltpu.SemaphoreType.DMA,
        ],
    )
    def k(flat_hbm, idx_hbm, out_hbm, idx_v, vals_v, sem):
        wid = jax.lax.axis_index("s") * nc + jax.lax.axis_index("c")
        base = wid * bpw
        pltpu.sync_copy(idx_hbm.at[pl.ds(base, bpw)], idx_v)
        pltpu.async_copy(flat_hbm.at[idx_v], vals_v, sem).wait()
        pltpu.sync_copy(vals_v, out_hbm.at[pl.ds(base, bpw)])

    return k(flat, flat_idx)


def _half_lse(x):
    m = jnp.max(x, axis=1)
    e = jnp.exp(x - m[:, None])
    ones = jnp.ones((_C, 1), jnp.float32)
    s = jax.lax.dot_general(
        e, ones, (((1,), (0,)), ((), ())),
        preferred_element_type=jnp.float32)[:, 0]
    return m + jnp.log(s)


def _lse_body(x1_ref, x2_ref, l1_ref, l2_ref):
    l1_ref[0, 0, :] = _half_lse(x1_ref[...])
    l2_ref[0, 0, :] = _half_lse(x2_ref[...])


def _select_body(l1_ref, l2_ref, xl_ref, out_ref):
    lse = jnp.concatenate(
        [l1_ref[...].reshape(_NB, _BR), l2_ref[...].reshape(_NB, _BR)],
        axis=0)
    loss = lse - xl_ref[...].reshape(_NS * _NB, _BR)
    i32_min = jnp.int32(_I32_MIN)
    i32_maxp = jnp.int32(_I32_MAXP)
    bits = jax.lax.bitcast_convert_type(loss, jnp.int32)
    key = jnp.where(bits < 0, bits ^ i32_maxp, bits)

    def body(t, prefix):
        lo = 30 - 2 * t
        c1 = prefix | jax.lax.shift_left(jnp.int32(1), lo)
        c2 = prefix | jax.lax.shift_left(jnp.int32(2), lo)
        c3 = prefix | jax.lax.shift_left(jnp.int32(3), lo)
        n1 = jnp.sum((key >= (c1 ^ i32_min)).astype(jnp.int32))
        n2 = jnp.sum((key >= (c2 ^ i32_min)).astype(jnp.int32))
        n3 = jnp.sum((key >= (c3 ^ i32_min)).astype(jnp.int32))
        d = ((n1 >= _K).astype(jnp.int32) + (n2 >= _K).astype(jnp.int32)
             + (n3 >= _K).astype(jnp.int32))
        return prefix | jax.lax.shift_left(d, lo)

    kth_biased = jax.lax.fori_loop(0, 16, body, jnp.int32(0))
    kth = kth_biased ^ i32_min
    mask = (key >= kth).astype(jnp.float32)
    out_ref[...] = (jnp.sum(loss * mask) / jnp.sum(mask)).reshape(1, 1)


def kernel(output, labels):
    labels_i = labels.astype(jnp.int32)
    flat_idx = jnp.arange(_N, dtype=jnp.int32) * _C + labels_i
    xl = _sc_gather(output.reshape(-1), flat_idx)

    lse_shape = jax.ShapeDtypeStruct((_NB, 1, _BR), jnp.float32)
    l1, l2 = pl.pallas_call(
        _lse_body,
        grid=(_NB,),
        in_specs=[
            pl.BlockSpec((_BR, _C), lambda i: (i, 0)),
            pl.BlockSpec((_BR, _C), lambda i: (i + _NB, 0)),
        ],
        out_specs=[
            pl.BlockSpec((1, 1, _BR), lambda i: (i, 0, 0)),
            pl.BlockSpec((1, 1, _BR), lambda i: (i, 0, 0)),
        ],
        out_shape=[lse_shape, lse_shape],
        compiler_params=pltpu.CompilerParams(
            dimension_semantics=("arbitrary",)),
    )(output, output)

    xl_r = xl.reshape(_NS * _NB, 1, _BR)
    out = pl.pallas_call(
        _select_body,
        in_specs=[
            pl.BlockSpec((_NB, 1, _BR), lambda: (0, 0, 0)),
            pl.BlockSpec((_NB, 1, _BR), lambda: (0, 0, 0)),
            pl.BlockSpec((_NS * _NB, 1, _BR), lambda: (0, 0, 0)),
        ],
        out_specs=pl.BlockSpec((1, 1), lambda: (0, 0)),
        out_shape=jax.ShapeDtypeStruct((1, 1), jnp.float32),
    )(l1, l2, xl_r)
    return out[0, 0]


# speedup vs baseline: 2.0238x; 2.0238x over previous
import jax
import jax.numpy as jnp
from jax.experimental import pallas as pl
from jax.experimental.pallas import tpu as pltpu

_N = 16384
_C = 1000
_BR = 1024
_NS = 2
_NB = _N // (_BR * _NS)
_K = _N - 15565

_I32_MIN = -2147483648
_I32_MAXP = 2147483647


def _half_loss(x, lab):
    m = jnp.max(x, axis=1)
    e = jnp.exp(x - m[:, None])
    ones = jnp.ones((_C, 1), jnp.float32)
    s = jax.lax.dot_general(
        e, ones, (((1,), (0,)), ((), ())),
        preferred_element_type=jnp.float32)[:, 0]
    col = jax.lax.broadcasted_iota(jnp.int32, x.shape, 1)
    xl = jnp.sum(jnp.where(col == lab[:, None], x, 0.0), axis=1)
    return m + jnp.log(s) - xl


def _select(loss):
    i32_min = jnp.int32(_I32_MIN)
    i32_maxp = jnp.int32(_I32_MAXP)
    bits = jax.lax.bitcast_convert_type(loss, jnp.int32)
    key = jnp.where(bits < 0, bits ^ i32_maxp, bits)

    def body(t, prefix):
        lo = 30 - 2 * t
        c1 = prefix | jax.lax.shift_left(jnp.int32(1), lo)
        c2 = prefix | jax.lax.shift_left(jnp.int32(2), lo)
        c3 = prefix | jax.lax.shift_left(jnp.int32(3), lo)
        n1 = jnp.sum((key >= (c1 ^ i32_min)).astype(jnp.int32))
        n2 = jnp.sum((key >= (c2 ^ i32_min)).astype(jnp.int32))
        n3 = jnp.sum((key >= (c3 ^ i32_min)).astype(jnp.int32))
        d = ((n1 >= _K).astype(jnp.int32) + (n2 >= _K).astype(jnp.int32)
             + (n3 >= _K).astype(jnp.int32))
        return prefix | jax.lax.shift_left(d, lo)

    kth_biased = jax.lax.fori_loop(0, 16, body, jnp.int32(0))
    kth = kth_biased ^ i32_min
    mask = (key >= kth).astype(jnp.float32)
    return jnp.sum(loss * mask) / jnp.sum(mask)


def _body(x1_ref, x2_ref, lab1_ref, lab2_ref, out_ref, loss_ref):
    i = pl.program_id(0)
    loss_ref[pl.ds(i, 1), :] = _half_loss(
        x1_ref[...], lab1_ref[0, 0, :]).reshape(1, _BR)
    loss_ref[pl.ds(i + _NB, 1), :] = _half_loss(
        x2_ref[...], lab2_ref[0, 0, :]).reshape(1, _BR)

    @pl.when(i == _NB - 1)
    def _():
        out_ref[...] = _select(loss_ref[...]).reshape(1, 1)


def kernel(output, labels):
    labels_r = labels.astype(jnp.int32).reshape(_N // _BR, 1, _BR)
    out = pl.pallas_call(
        _body,
        grid=(_NB,),
        in_specs=[
            pl.BlockSpec((_BR, _C), lambda i: (i, 0)),
            pl.BlockSpec((_BR, _C), lambda i: (i + _NB, 0)),
            pl.BlockSpec((1, 1, _BR), lambda i: (i, 0, 0)),
            pl.BlockSpec((1, 1, _BR), lambda i: (i + _NB, 0, 0)),
        ],
        out_specs=pl.BlockSpec((1, 1), lambda i: (0, 0)),
        out_shape=jax.ShapeDtypeStruct((1, 1), jnp.float32),
        scratch_shapes=[pltpu.VMEM((_NS * _NB, _BR), jnp.float32)],
        compiler_params=pltpu.CompilerParams(
            dimension_semantics=("arbitrary",)),
    )(output, output, labels_r, labels_r)
    return out[0, 0]


# speedup vs baseline: 2.0325x; 1.0043x over previous
import jax
import jax.numpy as jnp
from jax.experimental import pallas as pl
from jax.experimental.pallas import tpu as pltpu

_N = 16384
_C = 1000
_BR = 1024
_NS = 2
_NB = _N // (_BR * _NS)
_K = _N - 15565

_I32_MIN = -2147483648
_I32_MAXP = 2147483647


def _half_loss(x, lab):
    m = jnp.max(x, axis=1)
    e = jnp.exp(x - m[:, None])
    ones = jnp.ones((_C, 1), jnp.float32)
    s = jax.lax.dot_general(
        e, ones, (((1,), (0,)), ((), ())),
        preferred_element_type=jnp.float32)[:, 0]
    col = jax.lax.broadcasted_iota(jnp.int32, x.shape, 1)
    xl = jnp.sum(jnp.where(col == lab[:, None], x, 0.0), axis=1)
    return m + jnp.log(s) - xl


def _select(loss):
    i32_min = jnp.int32(_I32_MIN)
    i32_maxp = jnp.int32(_I32_MAXP)
    bits = jax.lax.bitcast_convert_type(loss, jnp.int32)
    key = jnp.where(bits < 0, bits ^ i32_maxp, bits)

    def _digit(prefix, lo, nvals):
        flags = jnp.int32(0)
        for d in range(1, nvals):
            cand = prefix | jax.lax.shift_left(jnp.int32(d), lo)
            cnt = jnp.sum((key >= (cand ^ i32_min)).astype(jnp.int32))
            flags = flags + (cnt >= _K).astype(jnp.int32)
        return prefix | jax.lax.shift_left(flags, lo)

    def body(t, prefix):
        return _digit(prefix, 27 - 3 * t, 8)

    kth_biased = jax.lax.fori_loop(
        0, 10, body, _digit(jnp.int32(0), 30, 4))
    kth = kth_biased ^ i32_min
    mask = (key >= kth).astype(jnp.float32)
    return jnp.sum(loss * mask) / jnp.sum(mask)


def _body(x1_ref, x2_ref, lab1_ref, lab2_ref, out_ref, loss_ref):
    i = pl.program_id(0)
    loss_ref[pl.ds(i, 1), :] = _half_loss(
        x1_ref[...], lab1_ref[0, 0, :]).reshape(1, _BR)
    loss_ref[pl.ds(i + _NB, 1), :] = _half_loss(
        x2_ref[...], lab2_ref[0, 0, :]).reshape(1, _BR)

    @pl.when(i == _NB - 1)
    def _():
        out_ref[...] = _select(loss_ref[...]).reshape(1, 1)


def kernel(output, labels):
    labels_r = labels.astype(jnp.int32).reshape(_N // _BR, 1, _BR)
    out = pl.pallas_call(
        _body,
        grid=(_NB,),
        in_specs=[
            pl.BlockSpec((_BR, _C), lambda i: (i, 0)),
            pl.BlockSpec((_BR, _C), lambda i: (i + _NB, 0)),
            pl.BlockSpec((1, 1, _BR), lambda i: (i, 0, 0)),
            pl.BlockSpec((1, 1, _BR), lambda i: (i + _NB, 0, 0)),
        ],
        out_specs=pl.BlockSpec((1, 1), lambda i: (0, 0)),
        out_shape=jax.ShapeDtypeStruct((1, 1), jnp.float32),
        scratch_shapes=[pltpu.VMEM((_NS * _NB, _BR), jnp.float32)],
        compiler_params=pltpu.CompilerParams(
            dimension_semantics=("arbitrary",)),
    )(output, output, labels_r, labels_r)
    return out[0, 0]
